# baseline (speedup 1.0000x reference)
import jax
import jax.numpy as jnp
import numpy as np
from jax.experimental import pallas as pl


def _seg_softmax(logits, seg, n):
    m = jax.ops.segment_max(logits, seg, num_segments=n)
    ex = jnp.exp(logits - m[seg])
    s = jax.ops.segment_sum(ex, seg, num_segments=n)
    return ex / (s[seg] + 1e-16)


def _scale_kernel(a_ref, w_ref, o_ref, *, add_one):
    w = w_ref[0, :]
    ws = (w[0] + w[1] + w[2]) / 3.0
    if add_one:
        ws = 1.0 + ws
    o_ref[...] = a_ref[...] * ws


def kernel(x, e, u, edge_index, edge_attr, params):
    p = params
    src = edge_index[0]
    dst = edge_index[1]
    n = x.shape[0]
    H, DH = 4, 128

    m = jax.nn.relu(x[src] + edge_attr @ p['We1'] + p['be1'])
    agg = jax.ops.segment_sum(m, dst, num_segments=n)
    h = (1.0 + p['eps1']) * x + agg
    x1 = jax.nn.relu(h @ p['W1a'] + p['b1a']) @ p['W1b'] + p['b1b']

    m = jax.nn.relu(x1[src] + edge_attr @ p['We2'] + p['be2'])
    agg = jax.ops.segment_sum(m, dst, num_segments=n)
    h = (1.0 + p['eps2']) * x1 + agg
    x2 = jax.nn.relu(h @ p['W2a'] + p['b2a']) @ p['W2b'] + p['b2b']

    qn = (x2 @ p['Wq']).reshape(n, H, DH)
    kn = (x2 @ p['Wk']).reshape(n, H, DH)
    vn = (x2 @ p['Wv']).reshape(n, H, DH)
    head_outs = []
    for hh in range(H):
        eemb_h = edge_attr @ p['WeT'][:, hh * DH:(hh + 1) * DH]
        q_h = qn[:, hh, :][dst]
        k_h = kn[:, hh, :][src] + eemb_h
        v_h = vn[:, hh, :][src] + eemb_h
        logits_h = (q_h * k_h).sum(-1) / np.sqrt(DH)
        alpha_h = _seg_softmax(logits_h, dst, n)
        out_h = jax.ops.segment_sum(alpha_h[:, None] * v_h, dst, num_segments=n)
        head_outs.append(out_h)
    x3 = jnp.concatenate(head_outs, axis=1) + x2 @ p['Wskip'] + p['bskip']

    xl = x3 @ p['Wl']
    xr = x3 @ p['Wr']
    eg = edge_attr @ p['WeG']
    s = jax.nn.leaky_relu(xl[src] + xr[dst] + eg, 0.2) @ p['att']
    a = _seg_softmax(s, dst, n)
    x4 = jax.ops.segment_sum(a[:, None] * xl[src], dst, num_segments=n) + p['bg']

    import functools
    w3 = p['w3'].reshape(1, 3)
    NB = 1000
    x_out = pl.pallas_call(
        functools.partial(_scale_kernel, add_one=True),
        grid=(x4.shape[0] // NB,),
        in_specs=[pl.BlockSpec((NB, x4.shape[1]), lambda i: (i, 0)),
                  pl.BlockSpec((1, 3), lambda i: (0, 0))],
        out_specs=pl.BlockSpec((NB, x4.shape[1]), lambda i: (i, 0)),
        out_shape=jax.ShapeDtypeStruct(x4.shape, x4.dtype),
    )(x4, w3)
    EB = 8000
    e_out = pl.pallas_call(
        functools.partial(_scale_kernel, add_one=False),
        grid=(edge_attr.shape[0] // EB,),
        in_specs=[pl.BlockSpec((EB, edge_attr.shape[1]), lambda i: (i, 0)),
                  pl.BlockSpec((1, 3), lambda i: (0, 0))],
        out_specs=pl.BlockSpec((EB, edge_attr.shape[1]), lambda i: (i, 0)),
        out_shape=jax.ShapeDtypeStruct(edge_attr.shape, edge_attr.dtype),
    )(edge_attr, w3)
    return x_out, e_out


# speedup vs baseline: 1.0586x; 1.0586x over previous
import functools
import jax
import jax.numpy as jnp
import numpy as np
from jax import lax
from jax.experimental import pallas as pl
from jax.experimental.pallas import tpu as pltpu, tpu_sc as plsc

N, E, D = 10000, 320000, 128
NC, NS = 2, 16
NW = NC * NS
EPW = E // NW
B = 80
NCHUNK = EPW // B
NZ = 632
NZL = N - 15 * NZ


def _gine_agg_body(x_hbm, ee_hbm, src_hbm, dst_hbm, out_hbm,
                   acc, src_v, dst_v, xg, ee_v, sem):
    cid = lax.axis_index("c")
    sid = lax.axis_index("s")
    wid = cid * NS + sid

    def zrow(b, c):
        for j in range(8):
            xg[b, pl.ds(j * 16, 16)] = jnp.zeros((16,), jnp.float32)
        return c
    lax.fori_loop(0, B, zrow, 0)

    @pl.when(sid < 15)
    def _():
        for off in range(0, NZ, B):
            w = min(B, NZ - off)
            pltpu.sync_copy(xg.at[pl.ds(0, w), :],
                            acc.at[pl.ds(sid * NZ + off, w), :])

    @pl.when(sid == 15)
    def _():
        for off in range(0, NZL, B):
            w = min(B, NZL - off)
            pltpu.sync_copy(xg.at[pl.ds(0, w), :],
                            acc.at[pl.ds(15 * NZ + off, w), :])
    plsc.subcore_barrier()

    def chunk(g, c):
        base = wid * EPW + g * B
        pltpu.sync_copy(src_hbm.at[pl.ds(base, B)], src_v)
        pltpu.async_copy(x_hbm.at[src_v], xg, sem).wait()
        pltpu.sync_copy(ee_hbm.at[pl.ds(base, B), :], ee_v)
        pltpu.sync_copy(dst_hbm.at[pl.ds(base, B)], dst_v)

        def brow(b, cc):
            for j in range(8):
                s = pl.ds(j * 16, 16)
                ee_v[b, s] = jnp.maximum(xg[b, s] + ee_v[b, s], 0.0)
            return cc
        lax.fori_loop(0, B, brow, 0)
        pltpu.sync_copy(ee_v, acc.at[dst_v], add=True)
        return c
    lax.fori_loop(0, NCHUNK, chunk, 0)
    plsc.subcore_barrier()

    @pl.when(sid < 15)
    def _():
        pltpu.sync_copy(acc.at[pl.ds(sid * NZ, NZ), :],
                        out_hbm.at[cid, pl.ds(sid * NZ, NZ), :])

    @pl.when(sid == 15)
    def _():
        pltpu.sync_copy(acc.at[pl.ds(15 * NZ, NZL), :],
                        out_hbm.at[cid, pl.ds(15 * NZ, NZL), :])


def _gine_agg(x, ee, src, dst):
    mesh = plsc.VectorSubcoreMesh(core_axis_name="c", subcore_axis_name="s",
                                  num_cores=NC, num_subcores=NS)
    f = pl.kernel(
        _gine_agg_body,
        out_type=jax.ShapeDtypeStruct((NC, N, D), jnp.float32),
        mesh=mesh,
        scratch_types=[
            pltpu.VMEM_SHARED((N, D), jnp.float32),
            pltpu.VMEM((B,), jnp.int32),
            pltpu.VMEM((B,), jnp.int32),
            pltpu.VMEM((B, D), jnp.float32),
            pltpu.VMEM((B, D), jnp.float32),
            pltpu.SemaphoreType.DMA,
        ],
    )
    out = f(x, ee, src, dst)
    return out[0] + out[1]


_NEG = -1e30


def _iota16():
    return lax.iota(jnp.int32, 16)


def _shift_get(a, idx):
    return a.at[idx].get(mode='promise_in_bounds')


def _segmax_rmw(mt, keys16, vals16):
    iota = _iota16()
    k = keys16
    v = vals16
    for sh in (1, 2, 4, 8):
        pidx = jnp.maximum(iota - sh, 0)
        pk = _shift_get(k, pidx)
        pv = _shift_get(v, pidx)
        v = jnp.where(pk == k, jnp.maximum(v, pv), v)
    nk = _shift_get(k, jnp.minimum(iota + 1, 15))
    last = (k != nk) | (iota == 15)
    cur = plsc.load_gather(mt, [k])
    plsc.store_scatter(mt, [k], jnp.maximum(cur, v), mask=last)


def _segsum_scatter(st, keys16, vals16):
    iota = _iota16()
    k = keys16
    v = vals16
    for sh in (1, 2, 4, 8):
        pidx = jnp.maximum(iota - sh, 0)
        pk = _shift_get(k, pidx)
        pv = _shift_get(v, pidx)
        v = jnp.where((iota >= sh) & (pk == k), v + pv, v)
    nk = _shift_get(k, jnp.minimum(iota + 1, 15))
    last = (k != nk) | (iota == 15)
    plsc.addupdate_scatter(st, [k], v, mask=last)


B4 = 80
NCHUNK4 = EPW // B4


def _gat_s_body(xl_hbm, xr_hbm, eg_hbm, att_hbm, src_hbm, dst_hbm,
                s_out,
                src_v, dst_v, xlg, xrg, egv, sbuf, attv, sem):
    cid = lax.axis_index("c")
    sid = lax.axis_index("s")
    wid = cid * NS + sid
    iota = _iota16()
    pltpu.sync_copy(att_hbm, attv)

    def chunk(g, c):
        base = wid * EPW + g * B4
        pltpu.sync_copy(src_hbm.at[pl.ds(base, B4)], src_v)
        pltpu.sync_copy(dst_hbm.at[pl.ds(base, B4)], dst_v)
        pltpu.async_copy(xl_hbm.at[src_v], xlg, sem).wait()
        pltpu.async_copy(xr_hbm.at[dst_v], xrg, sem).wait()
        pltpu.sync_copy(eg_hbm.at[pl.ds(base, B4), :], egv)

        def grp(t, c2):
            tot = jnp.zeros((16,), jnp.float32)
            for bi in range(16):
                b = t * 16 + bi
                acc = jnp.zeros((16,), jnp.float32)
                for j in range(8):
                    sl = pl.ds(j * 16, 16)
                    tt = xlg[b, sl] + xrg[b, sl] + egv[b, sl]
                    tt = jnp.maximum(tt, 0.0) + 0.2 * jnp.minimum(tt, 0.0)
                    acc = acc + tt * attv[sl]
                for sh in (1, 2, 4, 8):
                    acc = acc + _shift_get(acc, iota ^ sh)
                tot = jnp.where(iota == bi, acc, tot)
            sbuf[pl.ds(t * 16, 16)] = tot
            return c2
        lax.fori_loop(0, B4 // 16, grp, 0)
        pltpu.sync_copy(sbuf.at[pl.ds(0, B4)], s_out.at[pl.ds(base, B4)])
        return c
    lax.fori_loop(0, NCHUNK4, chunk, 0)


def _segmax_body(s_hbm, dst_hbm, mt_out, sv, dst_v, mt):
    cid = lax.axis_index("c")
    sid = lax.axis_index("s")
    wid = cid * NS + sid
    iota = _iota16()

    def initr(i, c):
        mt[pl.ds(i * 16, 16)] = jnp.full((16,), _NEG, jnp.float32)
        return c
    lax.fori_loop(0, N // 16, initr, 0)

    def chunk(g, c):
        base = wid * EPW + g * B4
        pltpu.sync_copy(s_hbm.at[pl.ds(base, B4)], sv)
        pltpu.sync_copy(dst_hbm.at[pl.ds(base, B4)], dst_v)

        def grp(t, c2):
            d16 = dst_v[pl.ds(t * 16, 16)]
            s16 = sv[pl.ds(t * 16, 16)]
            sd, perm = plsc.sort_key_val(d16, iota)
            vals = _shift_get(s16, perm)
            _segmax_rmw(mt, sd, vals)
            return c2
        lax.fori_loop(0, B4 // 16, grp, 0)
        return c
    lax.fori_loop(0, NCHUNK4, chunk, 0)
    pltpu.sync_copy(mt, mt_out.at[pl.ds(wid * N, N)])


def _softnum_body(s_hbm, m_hbm, dst_hbm, p_out, st_out, sv, dst_v, mtv, st):
    cid = lax.axis_index("c")
    sid = lax.axis_index("s")
    wid = cid * NS + sid
    iota = _iota16()
    pltpu.sync_copy(m_hbm, mtv)

    def zst(i, c):
        st[pl.ds(i * 16, 16)] = jnp.zeros((16,), jnp.float32)
        return c
    lax.fori_loop(0, N // 16, zst, 0)

    def chunk(g, c):
        base = wid * EPW + g * B4
        pltpu.sync_copy(s_hbm.at[pl.ds(base, B4)], sv)
        pltpu.sync_copy(dst_hbm.at[pl.ds(base, B4)], dst_v)

        def grp(t, c2):
            d16 = dst_v[pl.ds(t * 16, 16)]
            s16 = sv[pl.ds(t * 16, 16)]
            mv = plsc.load_gather(mtv, [d16])
            p16 = jnp.exp(s16 - mv)
            sv[pl.ds(t * 16, 16)] = p16
            sd, perm = plsc.sort_key_val(d16, iota)
            ps = _shift_get(p16, perm)
            _segsum_scatter(st, sd, ps)
            return c2
        lax.fori_loop(0, B4 // 16, grp, 0)
        pltpu.sync_copy(sv.at[pl.ds(0, B4)], p_out.at[pl.ds(base, B4)])
        return c
    lax.fori_loop(0, NCHUNK4, chunk, 0)
    pltpu.sync_copy(st, st_out.at[pl.ds(wid * N, N)])


def _wsum_body(v_hbm, ev_hbm, p_hbm, src_hbm, dst_hbm, out_hbm,
               acc, src_v, dst_v, vg, pv, obuf, sem, *, with_e):
    cid = lax.axis_index("c")
    sid = lax.axis_index("s")
    wid = cid * NS + sid

    def zrow(b, c):
        for j in range(8):
            obuf[b, pl.ds(j * 16, 16)] = jnp.zeros((16,), jnp.float32)
        return c
    lax.fori_loop(0, B4, zrow, 0)

    @pl.when(sid < 15)
    def _():
        for off in range(0, NZ, B4):
            w = min(B4, NZ - off)
            pltpu.sync_copy(obuf.at[pl.ds(0, w), :],
                            acc.at[pl.ds(sid * NZ + off, w), :])

    @pl.when(sid == 15)
    def _():
        for off in range(0, NZL, B4):
            w = min(B4, NZL - off)
            pltpu.sync_copy(obuf.at[pl.ds(0, w), :],
                            acc.at[pl.ds(15 * NZ + off, w), :])
    plsc.subcore_barrier()

    def chunk(g, c):
        base = wid * EPW + g * B4
        pltpu.sync_copy(src_hbm.at[pl.ds(base, B4)], src_v)
        pltpu.sync_copy(dst_hbm.at[pl.ds(base, B4)], dst_v)
        pltpu.async_copy(v_hbm.at[src_v], vg, sem).wait()
        if with_e:
            pltpu.sync_copy(ev_hbm.at[pl.ds(base, B4), :], obuf)
        pltpu.sync_copy(p_hbm.at[pl.ds(base, B4)], pv)

        def grp(t, c2):
            p16 = pv[pl.ds(t * 16, 16)]
            for bi in range(16):
                b = t * 16 + bi
                pb = jnp.full((16,), p16[bi], jnp.float32)
                for j in range(8):
                    sl = pl.ds(j * 16, 16)
                    if with_e:
                        obuf[b, sl] = (vg[b, sl] + obuf[b, sl]) * pb
                    else:
                        obuf[b, sl] = vg[b, sl] * pb
            return c2
        lax.fori_loop(0, B4 // 16, grp, 0)
        pltpu.sync_copy(obuf, acc.at[dst_v], add=True)
        return c
    lax.fori_loop(0, NCHUNK4, chunk, 0)
    plsc.subcore_barrier()

    @pl.when(sid < 15)
    def _():
        pltpu.sync_copy(acc.at[pl.ds(sid * NZ, NZ), :],
                        out_hbm.at[cid, pl.ds(sid * NZ, NZ), :])

    @pl.when(sid == 15)
    def _():
        pltpu.sync_copy(acc.at[pl.ds(15 * NZ, NZL), :],
                        out_hbm.at[cid, pl.ds(15 * NZ, NZL), :])


def _mesh():
    return plsc.VectorSubcoreMesh(core_axis_name="c", subcore_axis_name="s",
                                  num_cores=NC, num_subcores=NS)


_NOPASS = None


def _seg_softmax_sc(s, src, dst):
    fm = pl.kernel(
        _segmax_body,
        out_type=jax.ShapeDtypeStruct((NW * N,), jnp.float32),
        mesh=_mesh(),
        compiler_params=pltpu.CompilerParams(needs_layout_passes=False),
        scratch_types=[
            pltpu.VMEM((B4,), jnp.float32),
            pltpu.VMEM((B4,), jnp.int32),
            pltpu.VMEM((N,), jnp.float32),
        ],
    )
    mt = fm(s, dst)
    m = jnp.max(mt.reshape(NW, N), axis=0)

    fp = pl.kernel(
        _softnum_body,
        out_type=(jax.ShapeDtypeStruct((E,), jnp.float32),
                  jax.ShapeDtypeStruct((NW * N,), jnp.float32)),
        mesh=_mesh(),
        compiler_params=pltpu.CompilerParams(needs_layout_passes=False),
        scratch_types=[
            pltpu.VMEM((B4,), jnp.float32),
            pltpu.VMEM((B4,), jnp.int32),
            pltpu.VMEM((N,), jnp.float32),
            pltpu.VMEM((N,), jnp.float32),
        ],
    )
    pvals, stv = fp(s, m, dst)
    ssum = jnp.sum(stv.reshape(NW, N), axis=0)
    return pvals, ssum


def _weighted_scatter(v, ev, pvals, src, dst, with_e):
    fw = pl.kernel(
        functools.partial(_wsum_body, with_e=with_e),
        out_type=jax.ShapeDtypeStruct((NC, N, D), jnp.float32),
        mesh=_mesh(),
        scratch_types=[
            pltpu.VMEM_SHARED((N, D), jnp.float32),
            pltpu.VMEM((B4,), jnp.int32),
            pltpu.VMEM((B4,), jnp.int32),
            pltpu.VMEM((B4, D), jnp.float32),
            pltpu.VMEM((B4,), jnp.float32),
            pltpu.VMEM((B4, D), jnp.float32),
            pltpu.SemaphoreType.DMA,
        ],
    )
    out = fw(v, ev, pvals, src, dst)
    return out[0] + out[1]


def _gat_conv(xl, xr, eg, att, src, dst):
    fs = pl.kernel(
        _gat_s_body,
        out_type=jax.ShapeDtypeStruct((E,), jnp.float32),
        mesh=_mesh(),
        scratch_types=[
            pltpu.VMEM((B4,), jnp.int32),
            pltpu.VMEM((B4,), jnp.int32),
            pltpu.VMEM((B4, D), jnp.float32),
            pltpu.VMEM((B4, D), jnp.float32),
            pltpu.VMEM((B4, D), jnp.float32),
            pltpu.VMEM((128,), jnp.float32),
            pltpu.VMEM((D,), jnp.float32),
            pltpu.SemaphoreType.DMA,
        ],
    )
    s = fs(xl, xr, eg, att, src, dst)
    n = xl.shape[0]
    m = jax.ops.segment_max(s, dst, num_segments=n)
    ex = jnp.exp(s - m[dst])
    ssum = jax.ops.segment_sum(ex, dst, num_segments=n)
    ov = jax.ops.segment_sum(ex[:, None] * xl[src], dst, num_segments=n)
    return ov, ssum


def _seg_softmax(logits, seg, n):
    m = jax.ops.segment_max(logits, seg, num_segments=n)
    ex = jnp.exp(logits - m[seg])
    s = jax.ops.segment_sum(ex, seg, num_segments=n)
    return ex / (s[seg] + 1e-16)


def _scale_kernel(a_ref, w_ref, o_ref, *, add_one):
    w = w_ref[0, :]
    ws = (w[0] + w[1] + w[2]) / 3.0
    if add_one:
        ws = 1.0 + ws
    o_ref[...] = a_ref[...] * ws


def kernel(x, e, u, edge_index, edge_attr, params):
    p = params
    src = edge_index[0].astype(jnp.int32)
    dst = edge_index[1].astype(jnp.int32)
    n = x.shape[0]
    H, DH = 4, 128

    ee1 = edge_attr @ p['We1'] + p['be1']
    agg = _gine_agg(x, ee1, src, dst)
    h = (1.0 + p['eps1']) * x + agg
    x1 = jax.nn.relu(h @ p['W1a'] + p['b1a']) @ p['W1b'] + p['b1b']

    ee2 = edge_attr @ p['We2'] + p['be2']
    agg = _gine_agg(x1, ee2, src, dst)
    h = (1.0 + p['eps2']) * x1 + agg
    x2 = jax.nn.relu(h @ p['W2a'] + p['b2a']) @ p['W2b'] + p['b2b']

    qn = (x2 @ p['Wq']).reshape(n, H, DH)
    kn = (x2 @ p['Wk']).reshape(n, H, DH)
    vn = (x2 @ p['Wv']).reshape(n, H, DH)
    head_outs = []
    for hh in range(H):
        eemb_h = edge_attr @ p['WeT'][:, hh * DH:(hh + 1) * DH]
        q_h = qn[:, hh, :][dst]
        k_h = kn[:, hh, :][src] + eemb_h
        v_h = vn[:, hh, :][src] + eemb_h
        logits_h = (q_h * k_h).sum(-1) / np.sqrt(DH)
        alpha_h = _seg_softmax(logits_h, dst, n)
        out_h = jax.ops.segment_sum(alpha_h[:, None] * v_h, dst, num_segments=n)
        head_outs.append(out_h)
    x3 = jnp.concatenate(head_outs, axis=1) + x2 @ p['Wskip'] + p['bskip']

    xl = x3 @ p['Wl']
    xr = x3 @ p['Wr']
    eg = edge_attr @ p['WeG']
    s = jax.nn.leaky_relu(xl[src] + xr[dst] + eg, 0.2) @ p['att']
    a = _seg_softmax(s, dst, n)
    x4 = jax.ops.segment_sum(a[:, None] * xl[src], dst, num_segments=n) + p['bg']

    w3 = p['w3'].reshape(1, 3)
    NB = 1000
    x_out = pl.pallas_call(
        functools.partial(_scale_kernel, add_one=True),
        grid=(x4.shape[0] // NB,),
        in_specs=[pl.BlockSpec((NB, x4.shape[1]), lambda i: (i, 0)),
                  pl.BlockSpec((1, 3), lambda i: (0, 0))],
        out_specs=pl.BlockSpec((NB, x4.shape[1]), lambda i: (i, 0)),
        out_shape=jax.ShapeDtypeStruct(x4.shape, x4.dtype),
    )(x4, w3)
    EB = 8000
    e_out = pl.pallas_call(
        functools.partial(_scale_kernel, add_one=False),
        grid=(edge_attr.shape[0] // EB,),
        in_specs=[pl.BlockSpec((EB, edge_attr.shape[1]), lambda i: (i, 0)),
                  pl.BlockSpec((1, 3), lambda i: (0, 0))],
        out_specs=pl.BlockSpec((EB, edge_attr.shape[1]), lambda i: (i, 0)),
        out_shape=jax.ShapeDtypeStruct(edge_attr.shape, edge_attr.dtype),
    )(edge_attr, w3)
    return x_out, e_out
